# baseline (speedup 1.0000x reference)
import functools

import jax
import jax.numpy as jnp
from jax import lax
from jax.experimental import pallas as pl
from jax.experimental.pallas import tpu as pltpu
from jax.experimental.pallas import tpu_sc as plsc

_VOCAB = 100000
_EMBED = 32
_B = 4096
_L = 200
_N = _B * _L

_NC = 2
_NS = 16
_NW = _NC * _NS
_PER_W = _N // _NW
_CHUNK = 1600
_NCHUNK = _PER_W // _CHUNK
_NPAIR = _NCHUNK // 2


def _emb_body(table_hbm, idx_hbm, out_hbm,
              idx_v0, idx_v1, rows_v0, rows_v1, sem0, sem1):
    wid = lax.axis_index("s") * _NC + lax.axis_index("c")
    base = wid * _PER_W
    bufs = ((idx_v0, rows_v0, sem0), (idx_v1, rows_v1, sem1))

    def load_and_fire(i, b):
        idx_v, rows_v, sem = bufs[b]
        pltpu.sync_copy(idx_hbm.at[pl.ds(base + i * _CHUNK, _CHUNK)], idx_v)
        pltpu.async_copy(table_hbm.at[idx_v], rows_v, sem)

    def drain_and_store(i, b):
        idx_v, rows_v, sem = bufs[b]
        pltpu.make_async_copy(table_hbm.at[idx_v], rows_v, sem).wait()
        pltpu.sync_copy(
            rows_v,
            out_hbm.at[pl.ds(base + i * _CHUNK, _CHUNK), pl.ds(0, _EMBED)])

    load_and_fire(0, 0)

    def pair(j, _):
        for b in range(2):
            i = 2 * j + b
            if b == 0:
                load_and_fire(i + 1, 1)
            else:
                @pl.when(j < _NPAIR - 1)
                def _():
                    load_and_fire(i + 1, 0)
            drain_and_store(i, b)
        return 0

    lax.fori_loop(0, _NPAIR, pair, 0)


_emb = functools.partial(
    pl.kernel,
    mesh=plsc.VectorSubcoreMesh(core_axis_name="c", subcore_axis_name="s"),
    out_type=jax.ShapeDtypeStruct((_N, 128), jnp.float32),
    scratch_types=[
        pltpu.VMEM((_CHUNK,), jnp.int32),
        pltpu.VMEM((_CHUNK,), jnp.int32),
        pltpu.VMEM((_CHUNK, _EMBED), jnp.float32),
        pltpu.VMEM((_CHUNK, _EMBED), jnp.float32),
        pltpu.SemaphoreType.DMA,
        pltpu.SemaphoreType.DMA,
    ],
    compiler_params=pltpu.CompilerParams(use_tc_tiling_on_sc=False),
)(_emb_body)


def kernel(indices, table):
    flat_idx = indices.reshape(_N)
    out = _emb(table, flat_idx)
    return out.reshape(_B, _L, 128)[:, :, :_EMBED]
